# baseline (speedup 1.0000x reference)
import functools

import jax
import jax.numpy as jnp
from jax import lax
from jax.experimental import pallas as pl
from jax.experimental.pallas import tpu as pltpu
from jax.experimental.pallas import tpu_sc as plsc

B = 32768
D = 512
NW = 32
BPW = B // NW
QPW = BPW // 4
CQ = 16
NCH = QPW // CQ
NB = 2
L = 16
QR = 88


@functools.partial(
    pl.kernel,
    mesh=plsc.VectorSubcoreMesh(core_axis_name="c", subcore_axis_name="s"),
    out_type=jax.ShapeDtypeStruct((B // 4, 4 * D), jnp.float32),
    compiler_params=pltpu.CompilerParams(needs_layout_passes=False),
    scratch_types=[
        pltpu.VMEM((8, 128), jnp.int32),
        pltpu.VMEM((NCH, CQ), jnp.int32),
        pltpu.VMEM((NB, CQ, 4 * D), jnp.float32),
        pltpu.VMEM((3 * D,), jnp.float32),
        pltpu.VMEM((8, 4 * D), jnp.float32),
        pltpu.HBM((2 * QR, 4 * D), jnp.float32),
        pltpu.SemaphoreType.DMA,
        pltpu.SemaphoreType.DMA,
        pltpu.SemaphoreType.DMA,
        pltpu.SemaphoreType.DMA,
        pltpu.SemaphoreType.DMA,
        pltpu.SemaphoreType.DMA,
    ],
)
def _emb(x_hbm, table_hbm, out_hbm, idx_v, gidx, buf, tab_v, stage_v,
         quadtab, g0, g1, g2, s0, s1, s2):
    gsems = (g0, g1, g2)
    ssems = (s0, s1, s2)
    core = lax.axis_index("c")
    sid = lax.axis_index("s")
    wid = sid * 2 + core
    qbase = wid * QPW
    iota = lax.iota(jnp.int32, L)

    pltpu.sync_copy(table_hbm, tab_v)
    pltpu.sync_copy(x_hbm.at[wid], idx_v)

    @pl.when(sid < 11)
    def _build():
        for k in range(8):
            q = 8 * sid + k
            for j in range(4):
                p = (27, 9, 3, 1)[j]
                digit = lax.rem(lax.div(q, p), 3)
                srow = digit * D
                for kk in range(D // L):
                    val = plsc.load_gather(tab_v, [srow + kk * L + iota])
                    stage_v[k, pl.ds(j * D + kk * L, L)] = val
        pltpu.sync_copy(
            stage_v,
            quadtab.at[pl.ds(pl.multiple_of(QR * core + 8 * sid, 8), 8)])

    cbase = core * QR
    for G in range(QPW // L):
        pos = 64 * G + 4 * iota
        xs = []
        for j in range(4):
            rows = (pos + j) // 128
            cols = (pos + j) % 128
            xs.append(plsc.load_gather(idx_v, [rows, cols]))
        qv = ((xs[0] * 3 + xs[1]) * 3 + xs[2]) * 3 + xs[3]
        gidx[G // (CQ // L) if CQ != L else G, :] = cbase + qv

    plsc.subcore_barrier()

    gd = [None] * NB
    sd = [None] * NB
    for b in range(NB):
        gd[b] = pltpu.async_copy(quadtab.at[gidx.at[b]], buf.at[b], gsems[b])
    for c in range(NCH):
        b = c % NB
        gd[b].wait()
        sd[b] = pltpu.async_copy(
            buf.at[b], out_hbm.at[pl.ds(qbase + c * CQ, CQ)], ssems[b])
        n = c - 1 + NB
        if c >= 1 and n < NCH:
            bm = (c - 1) % NB
            sd[bm].wait()
            gd[bm] = pltpu.async_copy(
                quadtab.at[gidx.at[n]], buf.at[bm], gsems[bm])
    for c in range(NCH - NB, NCH):
        sd[c % NB].wait()


def kernel(x, table):
    xw = x.reshape(NW, 8, 128).astype(jnp.int32)
    out = _emb(xw, table.reshape(-1).astype(jnp.float32))
    return out.reshape(x.shape + (table.shape[1],))


# speedup vs baseline: 2.1747x; 2.1747x over previous
import functools

import jax
import jax.numpy as jnp
from jax import lax
from jax.experimental import pallas as pl
from jax.experimental.pallas import tpu as pltpu
from jax.experimental.pallas import tpu_sc as plsc

B = 32768
D = 512
NW = 32
BPW = B // NW
CH = 64
NCH = BPW // CH
NB = 3
L = 16


@functools.partial(
    pl.kernel,
    mesh=plsc.VectorSubcoreMesh(core_axis_name="c", subcore_axis_name="s"),
    out_type=jax.ShapeDtypeStruct((B, D), jnp.float32),
    scratch_types=[
        pltpu.VMEM((NCH, CH), jnp.int32),
        pltpu.VMEM((NB, CH, D), jnp.float32),
        pltpu.VMEM((8, D), jnp.float32),
        pltpu.HBM((NW * 128, D), jnp.float32),
        pltpu.SemaphoreType.DMA,
        pltpu.SemaphoreType.DMA,
        pltpu.SemaphoreType.DMA,
        pltpu.SemaphoreType.DMA,
        pltpu.SemaphoreType.DMA,
        pltpu.SemaphoreType.DMA,
    ],
)
def _emb(x_hbm, table_hbm, out_hbm, idx_v, buf, tab_v, tabrep,
         g0, g1, g2, s0, s1, s2):
    gsems = (g0, g1, g2)
    ssems = (s0, s1, s2)
    wid = lax.axis_index("s") * 2 + lax.axis_index("c")
    base = wid * BPW

    pltpu.sync_copy(table_hbm, tab_v.at[pl.ds(0, 3)])
    for j in range(16):
        pltpu.sync_copy(tab_v, tabrep.at[pl.ds(wid * 128 + 8 * j, 8)])

    pltpu.sync_copy(x_hbm.at[wid], idx_v)
    off = wid * 128 + 8 * lax.iota(jnp.int32, L)
    for r in range(NCH):
        for k in range(CH // L):
            sl = (r, pl.ds(k * L, L))
            idx_v[sl] = idx_v[sl] + off

    gd = [None] * NB
    sd = [None] * NB
    for b in range(NB):
        gd[b] = pltpu.async_copy(tabrep.at[idx_v.at[b]], buf.at[b], gsems[b])
    for c in range(NCH):
        b = c % NB
        gd[b].wait()
        sd[b] = pltpu.async_copy(
            buf.at[b], out_hbm.at[pl.ds(base + c * CH, CH)], ssems[b])
        n = c - 1 + NB
        if c >= 1 and n < NCH:
            bm = (c - 1) % NB
            sd[bm].wait()
            gd[bm] = pltpu.async_copy(
                tabrep.at[idx_v.at[n]], buf.at[bm], gsems[bm])
    for c in range(NCH - NB, NCH):
        sd[c % NB].wait()


def kernel(x, table):
    xw = x.reshape(NW, NCH, CH).astype(jnp.int32)
    out = _emb(xw, table.astype(jnp.float32))
    return out.reshape(x.shape + (table.shape[1],))
